# Initial kernel scaffold; baseline (speedup 1.0000x reference)
import jax
import jax.numpy as jnp
from jax.experimental import pallas as pl


def kernel(feature_id, feature_val, W, bias):
    raise NotImplementedError("write your pallas kernel here")


# speedup vs baseline: 1.1054x; 1.1054x over previous
import functools

import jax
import jax.numpy as jnp
from jax import lax
from jax.experimental import pallas as pl
from jax.experimental.pallas import tpu as pltpu
from jax.experimental.pallas import tpu_sc as plsc

VOCAB = 1000000
BATCH = 16384
FIELDS = 26

NUM_WORKERS = 32
LANES = 16
BPW = BATCH // NUM_WORKERS
IPW = BPW * FIELDS
CHUNK = 128
NCHUNKS = IPW // CHUNK
INFLIGHT = 8


def _sc_body(fid_hbm, fval_hbm, w_hbm, out_hbm, idx_v, emb_v, val_v, out_v, sem):
    c = lax.axis_index("c")
    s = lax.axis_index("s")
    wid = s * 2 + c

    pltpu.sync_copy(fid_hbm.at[wid], idx_v)
    pltpu.sync_copy(fval_hbm.at[pl.ds(wid * IPW, IPW)], val_v)

    def gather_group(g, carry):
        base = g * INFLIGHT
        for b in range(INFLIGHT):
            pltpu.make_async_copy(
                w_hbm.at[idx_v.at[base + b]],
                emb_v.at[pl.ds((base + b) * CHUNK, CHUNK)],
                sem,
            ).start()
        for b in range(INFLIGHT):
            pltpu.make_async_copy(
                w_hbm.at[idx_v.at[base + b]],
                emb_v.at[pl.ds((base + b) * CHUNK, CHUNK)],
                sem,
            ).wait()
        return carry

    lax.fori_loop(0, NCHUNKS // INFLIGHT, gather_group, 0)

    lane = lax.iota(jnp.int32, LANES)

    def chunk_body(cc, carry):
        base = (lane + cc * LANES) * FIELDS
        acc = jnp.zeros((LANES,), jnp.float32)
        for j in range(FIELDS):
            e = plsc.load_gather(emb_v, [base + j])
            v = plsc.load_gather(val_v, [base + j])
            acc = acc + e * v
        out_v[pl.ds(cc * LANES, LANES)] = acc
        return carry

    lax.fori_loop(0, BPW // LANES, chunk_body, 0)

    pltpu.sync_copy(out_v, out_hbm.at[pl.ds(wid * BPW, BPW)])


_sc_kernel = functools.partial(
    pl.kernel,
    mesh=plsc.VectorSubcoreMesh(core_axis_name="c", subcore_axis_name="s"),
    out_type=jax.ShapeDtypeStruct((BATCH,), jnp.float32),
    scratch_types=[
        pltpu.VMEM((NCHUNKS, CHUNK), jnp.int32),
        pltpu.VMEM((IPW,), jnp.float32),
        pltpu.VMEM((IPW,), jnp.float32),
        pltpu.VMEM((BPW,), jnp.float32),
        pltpu.SemaphoreType.DMA,
    ],
    compiler_params=pltpu.CompilerParams(needs_layout_passes=False),
)(_sc_body)


@jax.jit
def kernel(feature_id, feature_val, W, bias):
    fid = feature_id.astype(jnp.int32).reshape(NUM_WORKERS, NCHUNKS, CHUNK)
    fval = feature_val.reshape(-1)
    w = W.reshape(-1)
    logit = _sc_kernel(fid, fval, w)
    return logit + bias


# speedup vs baseline: 1.1968x; 1.0827x over previous
import functools

import jax
import jax.numpy as jnp
from jax import lax
from jax.experimental import pallas as pl
from jax.experimental.pallas import tpu as pltpu
from jax.experimental.pallas import tpu_sc as plsc

VOCAB = 1000000
BATCH = 16384
FIELDS = 26

NUM_WORKERS = 32
LANES = 16
BPW = BATCH // NUM_WORKERS
IPW = BPW * FIELDS
CHUNK = 128
NCHUNKS = IPW // CHUNK
INFLIGHT = 8


def _sc_body(fid_hbm, fval_hbm, w_hbm, out_hbm, idx_v, emb_v, val_v, out_v, sem):
    c = lax.axis_index("c")
    s = lax.axis_index("s")
    wid = s * 2 + c

    pltpu.sync_copy(fid_hbm.at[pl.ds(wid * IPW, IPW)], idx_v)
    pltpu.sync_copy(fval_hbm.at[pl.ds(wid * IPW, IPW)], val_v)

    pltpu.make_async_copy(w_hbm.at[idx_v], emb_v, sem).start()
    pltpu.make_async_copy(w_hbm.at[idx_v], emb_v, sem).wait()

    lane = lax.iota(jnp.int32, LANES)

    def chunk_body(cc, carry):
        base = (lane + cc * LANES) * FIELDS
        acc = jnp.zeros((LANES,), jnp.float32)
        for j in range(FIELDS):
            e = plsc.load_gather(emb_v, [base + j])
            v = plsc.load_gather(val_v, [base + j])
            acc = acc + e * v
        out_v[pl.ds(cc * LANES, LANES)] = acc
        return carry

    lax.fori_loop(0, BPW // LANES, chunk_body, 0)

    pltpu.sync_copy(out_v, out_hbm.at[pl.ds(wid * BPW, BPW)])


_sc_kernel = functools.partial(
    pl.kernel,
    mesh=plsc.VectorSubcoreMesh(core_axis_name="c", subcore_axis_name="s"),
    out_type=jax.ShapeDtypeStruct((BATCH,), jnp.float32),
    scratch_types=[
        pltpu.VMEM((IPW,), jnp.int32),
        pltpu.VMEM((IPW,), jnp.float32),
        pltpu.VMEM((IPW,), jnp.float32),
        pltpu.VMEM((BPW,), jnp.float32),
        pltpu.SemaphoreType.DMA,
    ],
    compiler_params=pltpu.CompilerParams(needs_layout_passes=False),
)(_sc_body)


@jax.jit
def kernel(feature_id, feature_val, W, bias):
    fid = feature_id.astype(jnp.int32).reshape(-1)
    fval = feature_val.reshape(-1)
    w = W.reshape(-1)
    logit = _sc_kernel(fid, fval, w)
    return logit + bias


# speedup vs baseline: 1.5469x; 1.2925x over previous
import functools

import jax
import jax.numpy as jnp
from jax import lax
from jax.experimental import pallas as pl
from jax.experimental.pallas import tpu as pltpu
from jax.experimental.pallas import tpu_sc as plsc

VOCAB = 1000000
BATCH = 16384
FIELDS = 26

NUM_WORKERS = 32
LANES = 16
BPW = BATCH // NUM_WORKERS
IPW = BPW * FIELDS


def _sc_body(fid_hbm, fval_hbm, w_hbm, bias_hbm, out_hbm,
             idx_v, emb_v, val_v, out_v, bias_v, sem):
    c = lax.axis_index("c")
    s = lax.axis_index("s")
    wid = s * 2 + c
    base_b = wid * BPW

    for j in range(FIELDS):
        pltpu.make_async_copy(
            fid_hbm.at[pl.ds(j * BATCH + base_b, BPW)],
            idx_v.at[pl.ds(j * BPW, BPW)], sem).start()
    for j in range(FIELDS):
        pltpu.make_async_copy(
            fval_hbm.at[pl.ds(j * BATCH + base_b, BPW)],
            val_v.at[pl.ds(j * BPW, BPW)], sem).start()
    pltpu.sync_copy(bias_hbm, bias_v)
    for j in range(FIELDS):
        pltpu.make_async_copy(
            fid_hbm.at[pl.ds(j * BATCH + base_b, BPW)],
            idx_v.at[pl.ds(j * BPW, BPW)], sem).wait()

    gather = pltpu.make_async_copy(w_hbm.at[idx_v], emb_v, sem)
    gather.start()
    for j in range(FIELDS):
        pltpu.make_async_copy(
            fval_hbm.at[pl.ds(j * BATCH + base_b, BPW)],
            val_v.at[pl.ds(j * BPW, BPW)], sem).wait()
    gather.wait()

    zero = jnp.zeros((LANES,), jnp.int32)
    bias = plsc.load_gather(bias_v, [zero])

    def chunk_body(cc, carry):
        off = cc * LANES
        acc = bias
        for j in range(FIELDS):
            e = emb_v[pl.ds(j * BPW + off, LANES)]
            v = val_v[pl.ds(j * BPW + off, LANES)]
            acc = acc + e * v
        out_v[pl.ds(off, LANES)] = acc
        return carry

    lax.fori_loop(0, BPW // LANES, chunk_body, 0)

    pltpu.sync_copy(out_v, out_hbm.at[pl.ds(base_b, BPW)])


_sc_kernel = functools.partial(
    pl.kernel,
    mesh=plsc.VectorSubcoreMesh(core_axis_name="c", subcore_axis_name="s"),
    out_type=jax.ShapeDtypeStruct((BATCH,), jnp.float32),
    scratch_types=[
        pltpu.VMEM((IPW,), jnp.int32),
        pltpu.VMEM((IPW,), jnp.float32),
        pltpu.VMEM((IPW,), jnp.float32),
        pltpu.VMEM((BPW,), jnp.float32),
        pltpu.VMEM((1,), jnp.float32),
        pltpu.SemaphoreType.DMA,
    ],
    compiler_params=pltpu.CompilerParams(needs_layout_passes=False),
)(_sc_body)


@jax.jit
def kernel(feature_id, feature_val, W, bias):
    fid = feature_id.astype(jnp.int32).T.reshape(-1)
    fval = feature_val.T.reshape(-1)
    return _sc_kernel(fid, fval, W.reshape(-1), bias)


# speedup vs baseline: 3.1827x; 2.0575x over previous
import functools

import jax
import jax.numpy as jnp
from jax import lax
from jax.experimental import pallas as pl
from jax.experimental.pallas import tpu as pltpu
from jax.experimental.pallas import tpu_sc as plsc

VOCAB = 1000000
VOCAB_PAD = 1000448
BATCH = 16384
FIELDS = 26

NUM_WORKERS = 32
LANES = 16
BPW = BATCH // NUM_WORKERS
IPW = BPW * FIELDS


def _sc_body(fid_hbm, fval_hbm, w_hbm, bias_hbm, out_hbm,
             idx_v, emb_v, val_v, out_v, bias_v, sem):
    c = lax.axis_index("c")
    s = lax.axis_index("s")
    wid = s * 2 + c
    base_b = wid * BPW

    for j in range(FIELDS):
        pltpu.make_async_copy(
            fid_hbm.at[pl.ds(j * BATCH + base_b, BPW)],
            idx_v.at[pl.ds(j * BPW, BPW)], sem).start()
    for j in range(FIELDS):
        pltpu.make_async_copy(
            fval_hbm.at[pl.ds(j * BATCH + base_b, BPW)],
            val_v.at[pl.ds(j * BPW, BPW)], sem).start()
    pltpu.sync_copy(bias_hbm, bias_v)
    for j in range(FIELDS):
        pltpu.make_async_copy(
            fid_hbm.at[pl.ds(j * BATCH + base_b, BPW)],
            idx_v.at[pl.ds(j * BPW, BPW)], sem).wait()

    gather = pltpu.make_async_copy(w_hbm.at[0].at[idx_v], emb_v, sem)
    gather.start()
    for j in range(FIELDS):
        pltpu.make_async_copy(
            fval_hbm.at[pl.ds(j * BATCH + base_b, BPW)],
            val_v.at[pl.ds(j * BPW, BPW)], sem).wait()
    gather.wait()

    zero = jnp.zeros((LANES,), jnp.int32)
    bias = plsc.load_gather(bias_v, [zero])

    def chunk_body(cc, carry):
        off = cc * LANES
        acc = bias
        for j in range(FIELDS):
            e = emb_v[pl.ds(j * BPW + off, LANES)]
            v = val_v[pl.ds(j * BPW + off, LANES)]
            acc = acc + e * v
        out_v[pl.ds(off, LANES)] = acc
        return carry

    lax.fori_loop(0, BPW // LANES, chunk_body, 0)

    pltpu.sync_copy(out_v, out_hbm.at[pl.ds(base_b, BPW)])


_sc_kernel = functools.partial(
    pl.kernel,
    mesh=plsc.VectorSubcoreMesh(core_axis_name="c", subcore_axis_name="s"),
    out_type=jax.ShapeDtypeStruct((BATCH,), jnp.float32),
    scratch_types=[
        pltpu.VMEM((IPW,), jnp.int32),
        pltpu.VMEM((IPW,), jnp.float32),
        pltpu.VMEM((IPW,), jnp.float32),
        pltpu.VMEM((BPW,), jnp.float32),
        pltpu.VMEM((1,), jnp.float32),
        pltpu.SemaphoreType.DMA,
    ],
    compiler_params=pltpu.CompilerParams(needs_layout_passes=False),
)(_sc_body)


@jax.jit
def kernel(feature_id, feature_val, W, bias):
    fid = feature_id.astype(jnp.int32).T.reshape(-1)
    fval = feature_val.T.reshape(-1)
    return _sc_kernel(fid, fval, W.T, bias)


# speedup vs baseline: 4.0458x; 1.2712x over previous
import functools

import jax
import jax.numpy as jnp
from jax import lax
from jax.experimental import pallas as pl
from jax.experimental.pallas import tpu as pltpu
from jax.experimental.pallas import tpu_sc as plsc

VOCAB = 1000000
VOCAB_PAD = 1000448
BATCH = 16384
FIELDS = 26

NUM_WORKERS = 32
LANES = 16
BPW = BATCH // NUM_WORKERS
IPW = BPW * FIELDS


W_SLICE = 62592
W_LAST = VOCAB - 15 * W_SLICE


def _sc_body(fid_hbm, fval_hbm, w_hbm, bias_hbm, out_hbm,
             idx_v, emb_v, val_v, out_v, bias_v, w_sh, sem, wsem):
    c = lax.axis_index("c")
    s = lax.axis_index("s")
    wid = s * 2 + c
    base_b = wid * BPW

    w_off = s * W_SLICE

    @pl.when(s < 15)
    def _():
        pltpu.make_async_copy(
            w_hbm.at[:, pl.ds(w_off, W_SLICE)],
            w_sh.at[:, pl.ds(w_off, W_SLICE)], wsem).start()

    @pl.when(s == 15)
    def _():
        pltpu.make_async_copy(
            w_hbm.at[:, pl.ds(15 * W_SLICE, W_LAST)],
            w_sh.at[:, pl.ds(15 * W_SLICE, W_LAST)], wsem).start()

    for j in range(FIELDS):
        pltpu.make_async_copy(
            fid_hbm.at[pl.ds(j * BATCH + base_b, BPW)],
            idx_v.at[pl.ds(j * BPW, BPW)], sem).start()
    for j in range(FIELDS):
        pltpu.make_async_copy(
            fval_hbm.at[pl.ds(j * BATCH + base_b, BPW)],
            val_v.at[pl.ds(j * BPW, BPW)], sem).start()
    pltpu.sync_copy(bias_hbm, bias_v)
    for j in range(FIELDS):
        pltpu.make_async_copy(
            fid_hbm.at[pl.ds(j * BATCH + base_b, BPW)],
            idx_v.at[pl.ds(j * BPW, BPW)], sem).wait()

    @pl.when(s < 15)
    def _():
        pltpu.make_async_copy(
            w_hbm.at[:, pl.ds(w_off, W_SLICE)],
            w_sh.at[:, pl.ds(w_off, W_SLICE)], wsem).wait()

    @pl.when(s == 15)
    def _():
        pltpu.make_async_copy(
            w_hbm.at[:, pl.ds(15 * W_SLICE, W_LAST)],
            w_sh.at[:, pl.ds(15 * W_SLICE, W_LAST)], wsem).wait()

    plsc.subcore_barrier()

    gather = pltpu.make_async_copy(w_sh.at[0].at[idx_v], emb_v, sem)
    gather.start()
    for j in range(FIELDS):
        pltpu.make_async_copy(
            fval_hbm.at[pl.ds(j * BATCH + base_b, BPW)],
            val_v.at[pl.ds(j * BPW, BPW)], sem).wait()
    gather.wait()

    zero = jnp.zeros((LANES,), jnp.int32)
    bias = plsc.load_gather(bias_v, [zero])

    def chunk_body(cc, carry):
        off = cc * LANES
        acc = bias
        for j in range(FIELDS):
            e = emb_v[pl.ds(j * BPW + off, LANES)]
            v = val_v[pl.ds(j * BPW + off, LANES)]
            acc = acc + e * v
        out_v[pl.ds(off, LANES)] = acc
        return carry

    lax.fori_loop(0, BPW // LANES, chunk_body, 0)

    pltpu.sync_copy(out_v, out_hbm.at[pl.ds(base_b, BPW)])


_sc_kernel = functools.partial(
    pl.kernel,
    mesh=plsc.VectorSubcoreMesh(core_axis_name="c", subcore_axis_name="s"),
    out_type=jax.ShapeDtypeStruct((BATCH,), jnp.float32),
    scratch_types=[
        pltpu.VMEM((IPW,), jnp.int32),
        pltpu.VMEM((IPW,), jnp.float32),
        pltpu.VMEM((IPW,), jnp.float32),
        pltpu.VMEM((BPW,), jnp.float32),
        pltpu.VMEM((1,), jnp.float32),
        pltpu.VMEM_SHARED((1, VOCAB), jnp.float32),
        pltpu.SemaphoreType.DMA,
        pltpu.SemaphoreType.DMA,
    ],
    compiler_params=pltpu.CompilerParams(needs_layout_passes=False),
)(_sc_body)


@jax.jit
def kernel(feature_id, feature_val, W, bias):
    fid = feature_id.astype(jnp.int32).T.reshape(-1)
    fval = feature_val.T.reshape(-1)
    return _sc_kernel(fid, fval, W.T, bias)


# speedup vs baseline: 4.0733x; 1.0068x over previous
import functools

import jax
import jax.numpy as jnp
from jax import lax
from jax.experimental import pallas as pl
from jax.experimental.pallas import tpu as pltpu
from jax.experimental.pallas import tpu_sc as plsc

VOCAB = 1000000
VOCAB_PAD = 1000448
BATCH = 16384
FIELDS = 26

NUM_WORKERS = 32
LANES = 16
BPW = BATCH // NUM_WORKERS
IPW = BPW * FIELDS


W_SLICE = 62592
W_LAST = VOCAB - 15 * W_SLICE


def _sc_body(fid_hbm, fval_hbm, w_hbm, bias_hbm, out_hbm,
             idx_v, emb_v, val_v, out_v, bias_v, w_sh, sem, wsem):
    c = lax.axis_index("c")
    s = lax.axis_index("s")
    wid = s * 2 + c
    base_b = wid * BPW

    w_off = s * W_SLICE

    @pl.when(s < 15)
    def _():
        pltpu.make_async_copy(
            w_hbm.at[:, pl.ds(w_off, W_SLICE)],
            w_sh.at[:, pl.ds(w_off, W_SLICE)], wsem).start()

    @pl.when(s == 15)
    def _():
        pltpu.make_async_copy(
            w_hbm.at[:, pl.ds(15 * W_SLICE, W_LAST)],
            w_sh.at[:, pl.ds(15 * W_SLICE, W_LAST)], wsem).start()

    for j in range(FIELDS):
        pltpu.make_async_copy(
            fid_hbm.at[pl.ds(j * BATCH + base_b, BPW)],
            idx_v.at[pl.ds(j * BPW, BPW)], sem).start()
    for j in range(FIELDS):
        pltpu.make_async_copy(
            fval_hbm.at[pl.ds(j * BATCH + base_b, BPW)],
            val_v.at[pl.ds(j * BPW, BPW)], sem).start()
    pltpu.sync_copy(bias_hbm, bias_v)
    for j in range(FIELDS):
        pltpu.make_async_copy(
            fid_hbm.at[pl.ds(j * BATCH + base_b, BPW)],
            idx_v.at[pl.ds(j * BPW, BPW)], sem).wait()

    @pl.when(s < 15)
    def _():
        pltpu.make_async_copy(
            w_hbm.at[:, pl.ds(w_off, W_SLICE)],
            w_sh.at[:, pl.ds(w_off, W_SLICE)], wsem).wait()

    @pl.when(s == 15)
    def _():
        pltpu.make_async_copy(
            w_hbm.at[:, pl.ds(15 * W_SLICE, W_LAST)],
            w_sh.at[:, pl.ds(15 * W_SLICE, W_LAST)], wsem).wait()

    plsc.subcore_barrier()

    HALF = 13 * BPW
    g_a = pltpu.make_async_copy(
        w_sh.at[0].at[idx_v.at[pl.ds(0, HALF)]], emb_v.at[pl.ds(0, HALF)], sem)
    g_b = pltpu.make_async_copy(
        w_sh.at[0].at[idx_v.at[pl.ds(HALF, HALF)]],
        emb_v.at[pl.ds(HALF, HALF)], sem)
    g_a.start()
    g_b.start()
    for j in range(FIELDS):
        pltpu.make_async_copy(
            fval_hbm.at[pl.ds(j * BATCH + base_b, BPW)],
            val_v.at[pl.ds(j * BPW, BPW)], sem).wait()

    zero = jnp.zeros((LANES,), jnp.int32)
    bias = plsc.load_gather(bias_v, [zero])
    g_a.wait()

    def first_half(cc, carry):
        off = cc * LANES
        acc = bias
        for j in range(13):
            e = emb_v[pl.ds(j * BPW + off, LANES)]
            v = val_v[pl.ds(j * BPW + off, LANES)]
            acc = acc + e * v
        out_v[pl.ds(off, LANES)] = acc
        return carry

    lax.fori_loop(0, BPW // LANES, first_half, 0)
    g_b.wait()

    def second_half(cc, carry):
        off = cc * LANES
        acc = out_v[pl.ds(off, LANES)]
        for j in range(13, FIELDS):
            e = emb_v[pl.ds(j * BPW + off, LANES)]
            v = val_v[pl.ds(j * BPW + off, LANES)]
            acc = acc + e * v
        out_v[pl.ds(off, LANES)] = acc
        return carry

    lax.fori_loop(0, BPW // LANES, second_half, 0)

    pltpu.sync_copy(out_v, out_hbm.at[pl.ds(base_b, BPW)])


_sc_kernel = functools.partial(
    pl.kernel,
    mesh=plsc.VectorSubcoreMesh(core_axis_name="c", subcore_axis_name="s"),
    out_type=jax.ShapeDtypeStruct((BATCH,), jnp.float32),
    scratch_types=[
        pltpu.VMEM((IPW,), jnp.int32),
        pltpu.VMEM((IPW,), jnp.float32),
        pltpu.VMEM((IPW,), jnp.float32),
        pltpu.VMEM((BPW,), jnp.float32),
        pltpu.VMEM((1,), jnp.float32),
        pltpu.VMEM_SHARED((1, VOCAB), jnp.float32),
        pltpu.SemaphoreType.DMA,
        pltpu.SemaphoreType.DMA,
    ],
    compiler_params=pltpu.CompilerParams(needs_layout_passes=False),
)(_sc_body)


@jax.jit
def kernel(feature_id, feature_val, W, bias):
    fid = feature_id.astype(jnp.int32).T.reshape(-1)
    fval = feature_val.T.reshape(-1)
    return _sc_kernel(fid, fval, W.T, bias)


# speedup vs baseline: 4.1198x; 1.0114x over previous
import functools

import jax
import jax.numpy as jnp
from jax import lax
from jax.experimental import pallas as pl
from jax.experimental.pallas import tpu as pltpu
from jax.experimental.pallas import tpu_sc as plsc

VOCAB = 1000000
BATCH = 16384
FIELDS = 26

NUM_WORKERS = 32
LANES = 16
BPW = BATCH // NUM_WORKERS
IPW = BPW * FIELDS

W_SLICE = 62592
W_LAST = VOCAB - 15 * W_SLICE

GROUPS = ((0, 7), (7, 13), (13, 20), (20, 26))


def _sc_body(fid_hbm, fval_hbm, w_hbm, bias_hbm, out_hbm,
             idx_v, emb_v, val_v, out_v, bias_v, w_sh, sem, wsem):
    c = lax.axis_index("c")
    s = lax.axis_index("s")
    wid = s * 2 + c
    base_b = wid * BPW

    w_off = s * W_SLICE

    @pl.when(s < 15)
    def _():
        pltpu.make_async_copy(
            w_hbm.at[:, pl.ds(w_off, W_SLICE)],
            w_sh.at[:, pl.ds(w_off, W_SLICE)], wsem).start()

    @pl.when(s == 15)
    def _():
        pltpu.make_async_copy(
            w_hbm.at[:, pl.ds(15 * W_SLICE, W_LAST)],
            w_sh.at[:, pl.ds(15 * W_SLICE, W_LAST)], wsem).start()

    def stage(j, carry):
        pltpu.make_async_copy(
            fid_hbm.at[pl.ds(j * BATCH + base_b, BPW)],
            idx_v.at[pl.ds(j * BPW, BPW)], sem).start()
        pltpu.make_async_copy(
            fval_hbm.at[pl.ds(j * BATCH + base_b, BPW)],
            val_v.at[pl.ds(j * BPW, BPW)], sem).start()
        return carry

    lax.fori_loop(0, FIELDS, stage, 0)
    pltpu.sync_copy(bias_hbm, bias_v)

    def drain(j, carry):
        pltpu.make_async_copy(
            fid_hbm.at[pl.ds(j * BATCH + base_b, BPW)],
            idx_v.at[pl.ds(j * BPW, BPW)], sem).wait()
        pltpu.make_async_copy(
            fval_hbm.at[pl.ds(j * BATCH + base_b, BPW)],
            val_v.at[pl.ds(j * BPW, BPW)], sem).wait()
        return carry

    lax.fori_loop(0, FIELDS, drain, 0)

    @pl.when(s < 15)
    def _():
        pltpu.make_async_copy(
            w_hbm.at[:, pl.ds(w_off, W_SLICE)],
            w_sh.at[:, pl.ds(w_off, W_SLICE)], wsem).wait()

    @pl.when(s == 15)
    def _():
        pltpu.make_async_copy(
            w_hbm.at[:, pl.ds(15 * W_SLICE, W_LAST)],
            w_sh.at[:, pl.ds(15 * W_SLICE, W_LAST)], wsem).wait()

    plsc.subcore_barrier()

    gathers = []
    for lo, hi in GROUPS:
        n = (hi - lo) * BPW
        g = pltpu.make_async_copy(
            w_sh.at[0].at[idx_v.at[pl.ds(lo * BPW, n)]],
            emb_v.at[pl.ds(lo * BPW, n)], sem)
        g.start()
        gathers.append(g)

    zero = jnp.zeros((LANES,), jnp.int32)
    bias = plsc.load_gather(bias_v, [zero])

    for gi, (lo, hi) in enumerate(GROUPS):
        gathers[gi].wait()

        def group_body(cc, carry, lo=lo, hi=hi, first=(gi == 0)):
            off = cc * LANES
            acc = bias if first else out_v[pl.ds(off, LANES)]
            for j in range(lo, hi):
                e = emb_v[pl.ds(j * BPW + off, LANES)]
                v = val_v[pl.ds(j * BPW + off, LANES)]
                acc = acc + e * v
            out_v[pl.ds(off, LANES)] = acc
            return carry

        lax.fori_loop(0, BPW // LANES, group_body, 0)

    pltpu.sync_copy(out_v, out_hbm.at[pl.ds(base_b, BPW)])


_sc_kernel = functools.partial(
    pl.kernel,
    mesh=plsc.VectorSubcoreMesh(core_axis_name="c", subcore_axis_name="s"),
    out_type=jax.ShapeDtypeStruct((BATCH,), jnp.float32),
    scratch_types=[
        pltpu.VMEM((IPW,), jnp.int32),
        pltpu.VMEM((IPW,), jnp.float32),
        pltpu.VMEM((IPW,), jnp.float32),
        pltpu.VMEM((BPW,), jnp.float32),
        pltpu.VMEM((1,), jnp.float32),
        pltpu.VMEM_SHARED((1, VOCAB), jnp.float32),
        pltpu.SemaphoreType.DMA,
        pltpu.SemaphoreType.DMA,
    ],
    compiler_params=pltpu.CompilerParams(needs_layout_passes=False),
)(_sc_body)


@jax.jit
def kernel(feature_id, feature_val, W, bias):
    fid = feature_id.astype(jnp.int32).T.reshape(-1)
    fval = feature_val.T.reshape(-1)
    return _sc_kernel(fid, fval, W.T, bias)
